# baseline (speedup 1.0000x reference)
import jax
import jax.numpy as jnp
from jax import lax
from jax.experimental import pallas as pl


def _vq_body(x_ref, pre_wT_ref, pre_b_ref, embT_ref, emb_ref, e_sq_ref,
             post_wT_ref, post_b_ref, z_ref, zq_ref, rec_ref):
    V = embT_ref.shape[1]
    x = x_ref[...]
    z = jnp.dot(x, pre_wT_ref[...],
                preferred_element_type=jnp.float32) + pre_b_ref[...]
    z_ref[...] = z
    z_sq = jnp.sum(z * z, axis=1, keepdims=True)
    m = jnp.dot(z, embT_ref[...], preferred_element_type=jnp.float32)
    dist = (z_sq + e_sq_ref[...]) - 2.0 * m
    dmin = jnp.min(dist, axis=1, keepdims=True)
    ids = lax.broadcasted_iota(jnp.int32, dist.shape, 1)
    tok = jnp.min(jnp.where(dist == dmin, ids, V), axis=1, keepdims=True)
    onehot = (ids == tok).astype(jnp.float32)
    zq = jnp.dot(onehot, emb_ref[...], preferred_element_type=jnp.float32)
    zq_ref[...] = zq
    rec_ref[...] = jnp.dot(zq, post_wT_ref[...],
                           preferred_element_type=jnp.float32) + post_b_ref[...]


def _make_call(N, C, E, V, T, interpret=False):
    grid = (N // T,)
    return pl.pallas_call(
        _vq_body,
        grid=grid,
        in_specs=[
            pl.BlockSpec((T, C), lambda i: (i, 0)),
            pl.BlockSpec((C, E), lambda i: (0, 0)),
            pl.BlockSpec((1, E), lambda i: (0, 0)),
            pl.BlockSpec((E, V), lambda i: (0, 0)),
            pl.BlockSpec((V, E), lambda i: (0, 0)),
            pl.BlockSpec((1, V), lambda i: (0, 0)),
            pl.BlockSpec((E, C), lambda i: (0, 0)),
            pl.BlockSpec((1, C), lambda i: (0, 0)),
        ],
        out_specs=[
            pl.BlockSpec((T, E), lambda i: (i, 0)),
            pl.BlockSpec((T, E), lambda i: (i, 0)),
            pl.BlockSpec((T, C), lambda i: (i, 0)),
        ],
        out_shape=[
            jax.ShapeDtypeStruct((N, E), jnp.float32),
            jax.ShapeDtypeStruct((N, E), jnp.float32),
            jax.ShapeDtypeStruct((N, C), jnp.float32),
        ],
        interpret=interpret,
    )


def kernel(x, pre_w, pre_b, emb, post_w, post_b):
    B, C, H, W = x.shape
    E = pre_w.shape[0]
    V = emb.shape[0]
    N = B * H * W
    T = 256
    x_flat = jnp.transpose(x, (0, 2, 3, 1)).reshape(N, C)
    e_sq = jnp.sum(emb ** 2, axis=1).reshape(1, V)
    call = _make_call(N, C, E, V, T)
    z_flat, zq_flat, rec_flat = call(
        x_flat, pre_w.T, pre_b.reshape(1, E), emb.T, emb, e_sq,
        post_w.T, post_b.reshape(1, C))
    z = z_flat.reshape(B, H, W, E).transpose(0, 3, 1, 2)
    z_q = zq_flat.reshape(B, H, W, E).transpose(0, 3, 1, 2)
    rec = rec_flat.reshape(B, H, W, C).transpose(0, 3, 1, 2)
    return (z, z_q, rec)


# speedup vs baseline: 1.5382x; 1.5382x over previous
import functools

import jax
import jax.numpy as jnp
from jax import lax
from jax.experimental import pallas as pl
from jax.experimental.pallas import tpu as pltpu
from jax.experimental.pallas import tpu_sc as plsc

_NC = 2
_NS = 16
_NW = _NC * _NS


def _vq_body(x_ref, pre_wT_ref, pre_b_ref, embT_ref, e_sq_ref, emb_ref,
             post_wT_ref, post_b_ref, z_ref, tok_ref, table_ref):
    V = embT_ref.shape[1]
    x = x_ref[...]
    z = jnp.dot(x, pre_wT_ref[...],
                preferred_element_type=jnp.float32) + pre_b_ref[...]
    z_ref[...] = z
    z_sq = jnp.sum(z * z, axis=1, keepdims=True)
    m = jnp.dot(z, embT_ref[...], preferred_element_type=jnp.float32)
    dist = (z_sq + e_sq_ref[...]) - 2.0 * m
    dmin = jnp.min(dist, axis=1, keepdims=True)
    ids = lax.broadcasted_iota(jnp.int32, dist.shape, 1)
    tok_ref[...] = jnp.min(jnp.where(dist == dmin, ids, V), axis=1,
                           keepdims=True)
    e_blk = emb_ref[...]
    rec_blk = jnp.dot(e_blk, post_wT_ref[...],
                      preferred_element_type=jnp.float32) + post_b_ref[...]
    pad = jnp.zeros((e_blk.shape[0], 32), jnp.float32)
    table_ref[...] = jnp.concatenate([e_blk, rec_blk, pad], axis=1)


def _make_tc_call(N, C, E, V, T):
    grid = (N // T,)
    vb = V // (N // T)
    return pl.pallas_call(
        _vq_body,
        grid=grid,
        in_specs=[
            pl.BlockSpec((T, C), lambda i: (i, 0)),
            pl.BlockSpec((C, E), lambda i: (0, 0)),
            pl.BlockSpec((1, E), lambda i: (0, 0)),
            pl.BlockSpec((E, V), lambda i: (0, 0)),
            pl.BlockSpec((1, V), lambda i: (0, 0)),
            pl.BlockSpec((vb, E), lambda i: (i, 0)),
            pl.BlockSpec((E, C), lambda i: (0, 0)),
            pl.BlockSpec((1, C), lambda i: (0, 0)),
        ],
        out_specs=[
            pl.BlockSpec((T, E), lambda i: (i, 0)),
            pl.BlockSpec((T, 1), lambda i: (i, 0)),
            pl.BlockSpec((vb, 128), lambda i: (i, 0)),
        ],
        out_shape=[
            jax.ShapeDtypeStruct((N, E), jnp.float32),
            jax.ShapeDtypeStruct((N, 1), jnp.int32),
            jax.ShapeDtypeStruct((V, 128), jnp.float32),
        ],
    )


def _make_sc_gather(V, N):
    b_per_w = N // _NW
    chunks = b_per_w // 128
    mesh = plsc.VectorSubcoreMesh(core_axis_name="c", subcore_axis_name="s")

    @functools.partial(
        pl.kernel, mesh=mesh,
        out_type=jax.ShapeDtypeStruct((N, 128), jnp.float32),
        scratch_types=[
            pltpu.VMEM((chunks, 128), jnp.int32),
            pltpu.VMEM((b_per_w, 128), jnp.float32),
            pltpu.SemaphoreType.DMA,
        ],
    )
    def k(table_hbm, idx_hbm, out_hbm, idx_v, rows_v, sem):
        wid = lax.axis_index("s") * _NC + lax.axis_index("c")
        base = wid * chunks
        pltpu.sync_copy(idx_hbm.at[pl.ds(base, chunks)], idx_v)
        copies = []
        for j in range(chunks):
            copies.append(pltpu.async_copy(
                table_hbm.at[idx_v.at[j]],
                rows_v.at[pl.ds(j * 128, 128)], sem))
        for c in copies:
            c.wait()
        pltpu.sync_copy(rows_v, out_hbm.at[pl.ds(wid * b_per_w, b_per_w)])

    return k


def kernel(x, pre_w, pre_b, emb, post_w, post_b):
    B, C, H, W = x.shape
    E = pre_w.shape[0]
    V = emb.shape[0]
    N = B * H * W
    T = 256
    x_flat = jnp.transpose(x, (0, 2, 3, 1)).reshape(N, C)
    e_sq = jnp.sum(emb ** 2, axis=1).reshape(1, V)
    z_flat, tok, table = _make_tc_call(N, C, E, V, T)(
        x_flat, pre_w.T, pre_b.reshape(1, E), emb.T, e_sq, emb,
        post_w.T, post_b.reshape(1, C))
    gathered = _make_sc_gather(V, N)(table, tok.reshape(N // 128, 128))
    zq_flat = gathered[:, :E]
    rec_flat = gathered[:, E:E + C]
    z = z_flat.reshape(B, H, W, E).transpose(0, 3, 1, 2)
    z_q = zq_flat.reshape(B, H, W, E).transpose(0, 3, 1, 2)
    rec = rec_flat.reshape(B, H, W, C).transpose(0, 3, 1, 2)
    return (z, z_q, rec)
